# baseline (speedup 1.0000x reference)
import functools

import jax
import jax.numpy as jnp
from jax import lax
from jax.experimental import pallas as pl
from jax.experimental.pallas import tpu as pltpu
from jax.experimental.pallas import tpu_sc as plsc

_CHUNK = 16
_NBUF = 6


@functools.lru_cache(maxsize=None)
def _build(V, D, N, chunk, nbuf):
    info = plsc.get_sparse_core_info()
    NC, NS = info.num_cores, info.num_subcores
    NW = NC * NS
    rows_per_w = N // NW
    n_chunks = rows_per_w // chunk
    mesh = plsc.VectorSubcoreMesh(core_axis_name="c", subcore_axis_name="s")

    @functools.partial(
        pl.kernel,
        mesh=mesh,
        out_type=jax.ShapeDtypeStruct((N, D), jnp.float32),
        scratch_types=[pltpu.VMEM((chunk,), jnp.int32)] * nbuf
        + [pltpu.VMEM((nbuf, chunk, D), jnp.float32)]
        + [pltpu.SemaphoreType.DMA] * (3 * nbuf),
    )
    def k(idx_hbm, table_hbm, out_hbm, *refs):
        idx_bufs = refs[:nbuf]
        rows_v = refs[nbuf]
        isem = refs[nbuf + 1:nbuf + 1 + nbuf]
        gsem = refs[nbuf + 1 + nbuf:nbuf + 1 + 2 * nbuf]
        ssem = refs[nbuf + 1 + 2 * nbuf:]
        wid = lax.axis_index("s") * NC + lax.axis_index("c")
        base = wid * rows_per_w
        idx0 = wid * n_chunks
        iloads = [None] * nbuf
        gets = [None] * nbuf
        puts = [None] * nbuf
        for j in range(min(nbuf, n_chunks)):
            iloads[j] = pltpu.async_copy(
                idx_hbm.at[idx0 + j], idx_bufs[j], isem[j])
        for j in range(min(nbuf, n_chunks)):
            iloads[j].wait()
            gets[j] = pltpu.async_copy(
                table_hbm.at[idx_bufs[j]], rows_v.at[j], gsem[j])
        for j in range(n_chunks):
            b = j % nbuf
            jn = j + nbuf
            gets[b].wait()
            puts[b] = pltpu.async_copy(
                rows_v.at[b], out_hbm.at[pl.ds(base + j * chunk, chunk)],
                ssem[b])
            if jn < n_chunks:
                iloads[b] = pltpu.async_copy(
                    idx_hbm.at[idx0 + jn], idx_bufs[b], isem[b])
                puts[b].wait()
                iloads[b].wait()
                gets[b] = pltpu.async_copy(
                    table_hbm.at[idx_bufs[b]], rows_v.at[b], gsem[b])
        for j in range(max(0, n_chunks - nbuf), n_chunks):
            puts[j % nbuf].wait()

    return k


def kernel(positions, embedding):
    B, T = positions.shape
    V, D = embedding.shape
    N = B * T
    idx = positions.reshape(N // _CHUNK, _CHUNK).astype(jnp.int32)
    half = N // 2
    hr = half // _CHUNK
    out0 = _build(V, D, half, _CHUNK, _NBUF)(idx[:hr], embedding)
    out1 = _build(V, D, half, _CHUNK, _NBUF)(idx[hr:], embedding)
    out = jnp.concatenate([out0, out1], axis=0)
    return out.reshape(B, T, D)


# speedup vs baseline: 1.7904x; 1.7904x over previous
import functools

import jax
import jax.numpy as jnp
from jax import lax
from jax.experimental import pallas as pl
from jax.experimental.pallas import tpu as pltpu
from jax.experimental.pallas import tpu_sc as plsc

_CHUNK = 32
_NBUF = 3


@functools.lru_cache(maxsize=None)
def _build(V, D, N, chunk, nbuf):
    info = plsc.get_sparse_core_info()
    NC, NS = info.num_cores, info.num_subcores
    NW = NC * NS
    rows_per_w = N // NW
    n_chunks = rows_per_w // chunk
    mesh = plsc.VectorSubcoreMesh(core_axis_name="c", subcore_axis_name="s")

    @functools.partial(
        pl.kernel,
        mesh=mesh,
        out_type=jax.ShapeDtypeStruct((N, D), jnp.float32),
        scratch_types=[pltpu.VMEM((chunk,), jnp.int32)] * nbuf
        + [pltpu.VMEM((nbuf, chunk, D), jnp.float32)]
        + [pltpu.SemaphoreType.DMA] * (3 * nbuf),
    )
    def k(idx_hbm, table_hbm, out_hbm, *refs):
        idx_bufs = refs[:nbuf]
        rows_v = refs[nbuf]
        isem = refs[nbuf + 1:nbuf + 1 + nbuf]
        gsem = refs[nbuf + 1 + nbuf:nbuf + 1 + 2 * nbuf]
        ssem = refs[nbuf + 1 + 2 * nbuf:]
        wid = lax.axis_index("s") * NC + lax.axis_index("c")
        base = wid * rows_per_w
        idx0 = wid * n_chunks
        iloads = [None] * nbuf
        gets = [None] * nbuf
        puts = [None] * nbuf
        for j in range(min(nbuf, n_chunks)):
            iloads[j] = pltpu.async_copy(
                idx_hbm.at[idx0 + j], idx_bufs[j], isem[j])
        for j in range(min(nbuf, n_chunks)):
            iloads[j].wait()
            gets[j] = pltpu.async_copy(
                table_hbm.at[idx_bufs[j]], rows_v.at[j], gsem[j])
        for j in range(n_chunks):
            b = j % nbuf
            jn = j + nbuf
            gets[b].wait()
            puts[b] = pltpu.async_copy(
                rows_v.at[b], out_hbm.at[pl.ds(base + j * chunk, chunk)],
                ssem[b])
            if jn < n_chunks:
                iloads[b] = pltpu.async_copy(
                    idx_hbm.at[idx0 + jn], idx_bufs[b], isem[b])
                puts[b].wait()
                iloads[b].wait()
                gets[b] = pltpu.async_copy(
                    table_hbm.at[idx_bufs[b]], rows_v.at[b], gsem[b])
        for j in range(max(0, n_chunks - nbuf), n_chunks):
            puts[j % nbuf].wait()

    return k


def kernel(positions, embedding):
    B, T = positions.shape
    V, D = embedding.shape
    N = B * T
    idx = positions.reshape(N // _CHUNK, _CHUNK).astype(jnp.int32)
    out = _build(V, D, N, _CHUNK, _NBUF)(idx, embedding)
    return out.reshape(B, T, D)
